# baseline (speedup 1.0000x reference)
import functools

import jax
import jax.numpy as jnp
from jax import lax
from jax.experimental import pallas as pl
from jax.experimental.pallas import tpu as pltpu
from jax.experimental.pallas import tpu_sc as plsc

D_MODEL = 1024
HIDDEN = 2816
NE = 8
NW = 32
ROW_BLOCK = 512
RB_SHIFT = ROW_BLOCK.bit_length() - 1


def _wid_base(t_per_w):
    cid = lax.axis_index("c")
    sid = lax.axis_index("s")
    wid = sid * 2 + cid
    return wid, wid * t_per_w


def _make_route(s_tokens):
    T = s_tokens // NW
    mesh = plsc.VectorSubcoreMesh(core_axis_name="c", subcore_axis_name="s")

    @functools.partial(
        pl.kernel, mesh=mesh,
        compiler_params=pltpu.CompilerParams(needs_layout_passes=False),
        out_type=[
            jax.ShapeDtypeStruct((2, s_tokens), jnp.int32),
            jax.ShapeDtypeStruct((2, s_tokens), jnp.float32),
            jax.ShapeDtypeStruct((NW, 16), jnp.int32),
        ],
        scratch_types=[
            pltpu.VMEM((NE, T), jnp.float32),
            pltpu.VMEM((2, T), jnp.int32),
            pltpu.VMEM((2, T), jnp.float32),
            pltpu.VMEM((16,), jnp.int32),
        ],
    )
    def route(lt_hbm, e_hbm, w_hbm, hist_hbm, lt_v, e_v, w_v, hist_v):
        wid, base = _wid_base(T)
        for e in range(NE):
            pltpu.sync_copy(lt_hbm.at[e, pl.ds(base, T)], lt_v.at[e])
        lanes = lax.iota(jnp.int32, 16)
        hist = jnp.zeros((16,), jnp.int32)
        for ch in range(T // 16):
            sl = pl.ds(ch * 16, 16)
            ls = [lt_v[e, sl] for e in range(NE)]
            m1 = ls[0]
            i1 = jnp.zeros((16,), jnp.int32)
            neg = jnp.full((16,), -1e30, jnp.float32)
            for e in range(1, NE):
                gt = ls[e] > m1
                m1 = jnp.where(gt, ls[e], m1)
                i1 = jnp.where(gt, jnp.full((16,), e, jnp.int32), i1)
            m2 = neg
            i2 = jnp.zeros((16,), jnp.int32)
            for e in range(NE):
                cand = jnp.where(i1 == e, neg, ls[e])
                gt = cand > m2
                m2 = jnp.where(gt, cand, m2)
                i2 = jnp.where(gt, jnp.full((16,), e, jnp.int32), i2)
            w1 = 1.0 / (1.0 + jnp.exp(m2 - m1))
            e_v[0, sl] = i1
            e_v[1, sl] = i2
            w_v[0, sl] = w1
            w_v[1, sl] = 1.0 - w1
            zero16 = jnp.zeros((16,), jnp.int32)
            for e in range(NE):
                c1 = plsc.all_reduce_population_count(i1 == e)
                c2 = plsc.all_reduce_population_count(i2 == e)
                hist = hist + jnp.where(lanes == e, c1 + c2, zero16)
        hist_v[...] = hist
        for s_ in range(2):
            pltpu.sync_copy(e_v.at[s_], e_hbm.at[s_, pl.ds(base, T)])
            pltpu.sync_copy(w_v.at[s_], w_hbm.at[s_, pl.ds(base, T)])
        pltpu.sync_copy(hist_v, hist_hbm.at[wid])

    return route


def _make_dispatch(s_tokens, cap):
    T = s_tokens // NW
    mesh = plsc.VectorSubcoreMesh(core_axis_name="c", subcore_axis_name="s")

    @functools.partial(
        pl.kernel, mesh=mesh,
        compiler_params=pltpu.CompilerParams(needs_layout_passes=False),
        out_type=[
            jax.ShapeDtypeStruct((cap, D_MODEL), jnp.float32),
            jax.ShapeDtypeStruct((2, s_tokens), jnp.int32),
        ],
        scratch_types=[
            pltpu.VMEM((NW, 16), jnp.int32),
            pltpu.VMEM((2, T), jnp.int32),
            pltpu.VMEM((2, T), jnp.int32),
            pltpu.VMEM((2, 16, D_MODEL), jnp.float32),
            pltpu.SemaphoreType.DMA((2,)),
            pltpu.SemaphoreType.DMA((2,)),
            pltpu.SemaphoreType.DMA((2,)),
        ],
    )
    def dispatch(x_hbm, e_hbm, hist_hbm, xs_hbm, dest_hbm,
                 hist_v, e_v, d_v, xbuf, sem1, sem2, in_sem):
        wid, base = _wid_base(T)
        pltpu.sync_copy(hist_hbm, hist_v)
        for s_ in range(2):
            pltpu.sync_copy(e_hbm.at[s_, pl.ds(base, T)], e_v.at[s_])
        lanes = lax.iota(jnp.int32, 16)
        zero16 = jnp.zeros((16,), jnp.int32)
        wid_v = jnp.full((16,), wid, jnp.int32)
        counts = zero16
        pre = zero16
        for wp in range(NW):
            row = hist_v[wp]
            counts = counts + row
            before = jnp.full((16,), wp, jnp.int32) < wid_v
            pre = pre + jnp.where(before, row, zero16)
        pc = ((counts + (ROW_BLOCK - 1)) >> RB_SHIFT) << RB_SHIFT
        incl = plsc.cumsum(pc)
        base_vec = (incl - pc) + pre
        b = [jnp.full((16,), jnp.max(jnp.where(lanes == e, base_vec, zero16)),
                      jnp.int32)
             for e in range(NE)]
        r = [zero16 for _ in range(NE)]
        for s_ in range(2):
            for ch in range(T // 16):
                sl = pl.ds(ch * 16, 16)
                v = e_v[s_, sl]
                dch = zero16
                for e in range(NE):
                    m = v == e
                    csum = plsc.cumsum(m.astype(jnp.int32))
                    dch = jnp.where(m, b[e] + r[e] + csum - 1, dch)
                    r[e] = r[e] + plsc.all_reduce_population_count(m)
                d_v[s_, sl] = dch
        for s_ in range(2):
            pltpu.sync_copy(d_v.at[s_], dest_hbm.at[s_, pl.ds(base, T)])
        nch = T // 16
        in_cp = {}
        sc_cp = {}
        in_cp[0] = pltpu.async_copy(
            x_hbm.at[pl.ds(base, 16)], xbuf.at[0], in_sem.at[0])
        for ch in range(nch):
            bslot = ch % 2
            nbslot = 1 - bslot
            in_cp[ch].wait()
            sl = pl.ds(ch * 16, 16)
            sc_cp[ch] = (
                pltpu.async_copy(xbuf.at[bslot], xs_hbm.at[d_v[0, sl]],
                                 sem1.at[bslot]),
                pltpu.async_copy(xbuf.at[bslot], xs_hbm.at[d_v[1, sl]],
                                 sem2.at[bslot]),
            )
            if ch + 1 < nch:
                if ch >= 1:
                    for c in sc_cp[ch - 1]:
                        c.wait()
                in_cp[ch + 1] = pltpu.async_copy(
                    x_hbm.at[pl.ds(base + (ch + 1) * 16, 16)],
                    xbuf.at[nbslot], in_sem.at[nbslot])
        for c in sc_cp[nch - 2] + sc_cp[nch - 1]:
            c.wait()

    return dispatch


def _make_combine(s_tokens, cap):
    T = s_tokens // NW
    mesh = plsc.VectorSubcoreMesh(core_axis_name="c", subcore_axis_name="s")

    @functools.partial(
        pl.kernel, mesh=mesh,
        compiler_params=pltpu.CompilerParams(needs_layout_passes=False),
        out_type=jax.ShapeDtypeStruct((s_tokens, D_MODEL), jnp.float32),
        scratch_types=[
            pltpu.VMEM((2, T), jnp.int32),
            pltpu.VMEM((2, T), jnp.float32),
            pltpu.VMEM((2, 16, D_MODEL), jnp.float32),
            pltpu.VMEM((2, 16, D_MODEL), jnp.float32),
            pltpu.SemaphoreType.DMA((2,)),
            pltpu.SemaphoreType.DMA((2,)),
            pltpu.SemaphoreType.DMA((2,)),
        ],
    )
    def combine(ys_hbm, dest_hbm, w_hbm, out_hbm,
                d_v, w_v, y1, y2, sem1, sem2, out_sem):
        wid, base = _wid_base(T)
        for s_ in range(2):
            pltpu.sync_copy(dest_hbm.at[s_, pl.ds(base, T)], d_v.at[s_])
            pltpu.sync_copy(w_hbm.at[s_, pl.ds(base, T)], w_v.at[s_])
        nch = T // 16
        g_cp = {}
        o_cp = {}
        g_cp[0] = (
            pltpu.async_copy(ys_hbm.at[d_v[0, pl.ds(0, 16)]], y1.at[0],
                             sem1.at[0]),
            pltpu.async_copy(ys_hbm.at[d_v[1, pl.ds(0, 16)]], y2.at[0],
                             sem2.at[0]),
        )
        for ch in range(nch):
            bslot = ch % 2
            nbslot = 1 - bslot
            for c in g_cp[ch]:
                c.wait()
            if ch + 1 < nch:
                if ch >= 1:
                    o_cp[ch - 1].wait()
                sln = pl.ds((ch + 1) * 16, 16)
                g_cp[ch + 1] = (
                    pltpu.async_copy(ys_hbm.at[d_v[0, sln]], y1.at[nbslot],
                                     sem1.at[nbslot]),
                    pltpu.async_copy(ys_hbm.at[d_v[1, sln]], y2.at[nbslot],
                                     sem2.at[nbslot]),
                )
            sl = pl.ds(ch * 16, 16)
            wch1 = w_v[0, sl]
            wch2 = w_v[1, sl]
            w1s = [jnp.full((16,), wch1[t], jnp.float32) for t in range(16)]
            w2s = [jnp.full((16,), wch2[t], jnp.float32) for t in range(16)]

            def blk_fn(blk, _, bslot=bslot, w1s=w1s, w2s=w2s):
                bsl = pl.ds(blk * 16, 16)
                for t in range(16):
                    y1[bslot, t, bsl] = (w1s[t] * y1[bslot, t, bsl]
                                         + w2s[t] * y2[bslot, t, bsl])
                return 0

            lax.fori_loop(0, D_MODEL // 16, blk_fn, 0)
            o_cp[ch] = pltpu.async_copy(
                y1.at[bslot], out_hbm.at[pl.ds(base + ch * 16, 16)],
                out_sem.at[bslot])
        o_cp[nch - 2].wait()
        o_cp[nch - 1].wait()

    return combine


def _ffn_body(be_ref, run_ref, nreal_ref,
              xs_ref, wg_hbm, wu_hbm, wd_hbm, out_ref,
              wgb, wub, wdb, g_sem, u_sem, d_sem):
    i = pl.program_id(0)
    slot = 0

    def copies(e, s):
        return (
            pltpu.make_async_copy(wg_hbm.at[e], wgb.at[s], g_sem.at[s]),
            pltpu.make_async_copy(wu_hbm.at[e], wub.at[s], u_sem.at[s]),
            pltpu.make_async_copy(wd_hbm.at[e], wdb.at[s], d_sem.at[s]),
        )

    @pl.when((i == 0) | (run_ref[i] != run_ref[jnp.maximum(i - 1, 0)]))
    def _():
        for c in copies(be_ref[i], 0):
            c.start()
        for c in copies(be_ref[i], 0):
            c.wait()

    @pl.when(i < nreal_ref[0])
    def _():
        x = xs_ref[...]
        g = jax.lax.dot_general(x, wgb[slot], (((1,), (1,)), ((), ())),
                                preferred_element_type=jnp.float32)
        u = jax.lax.dot_general(x, wub[slot], (((1,), (1,)), ((), ())),
                                preferred_element_type=jnp.float32)
        h = g * jax.nn.sigmoid(g) * u
        out_ref[...] = jax.lax.dot_general(h, wdb[slot], (((1,), (1,)), ((), ())),
                                           preferred_element_type=jnp.float32)


def _grouped_ffn(xs, wg, wu, wd, block_e, run_idx, nreal, nb):
    grid_spec = pltpu.PrefetchScalarGridSpec(
        num_scalar_prefetch=3,
        grid=(nb,),
        in_specs=[
            pl.BlockSpec((ROW_BLOCK, D_MODEL), lambda i, be, run, nr: (i, 0)),
            pl.BlockSpec(memory_space=pltpu.HBM),
            pl.BlockSpec(memory_space=pltpu.HBM),
            pl.BlockSpec(memory_space=pltpu.HBM),
        ],
        out_specs=pl.BlockSpec((ROW_BLOCK, D_MODEL), lambda i, be, run, nr: (i, 0)),
        scratch_shapes=[
            pltpu.VMEM((1, HIDDEN, D_MODEL), jnp.float32),
            pltpu.VMEM((1, HIDDEN, D_MODEL), jnp.float32),
            pltpu.VMEM((1, D_MODEL, HIDDEN), jnp.float32),
            pltpu.SemaphoreType.DMA((2,)),
            pltpu.SemaphoreType.DMA((2,)),
            pltpu.SemaphoreType.DMA((2,)),
        ],
    )
    return pl.pallas_call(
        _ffn_body,
        grid_spec=grid_spec,
        out_shape=jax.ShapeDtypeStruct((xs.shape[0], D_MODEL), jnp.float32),
        compiler_params=pltpu.CompilerParams(
            dimension_semantics=("arbitrary",),
        ),
    )(block_e, run_idx, nreal, xs, wg, wu, wd)


def kernel(x, W_router, Wg, Wu, Wd):
    bsz, seq, d = x.shape
    flat = x.reshape(-1, d)
    s = flat.shape[0]
    cap = s * 2 + NE * ROW_BLOCK
    nb = cap // ROW_BLOCK

    logits = flat @ W_router.T
    lt = logits.T

    e_arr, w_arr, hist = _make_route(s)(lt)

    counts = jnp.sum(hist, axis=0)[:NE]
    nblocks_e = (counts + ROW_BLOCK - 1) // ROW_BLOCK
    block_e = jnp.sum(
        (jnp.arange(nb)[:, None] >= jnp.cumsum(nblocks_e)[None, :]), axis=1
    ).astype(jnp.int32)
    block_e = jnp.minimum(block_e, NE - 1)
    nreal = jnp.sum(nblocks_e).astype(jnp.int32).reshape((1,))
    be_ext = jnp.concatenate([block_e, block_e[-1:]])
    run_idx = jnp.concatenate([
        jnp.zeros((1,), jnp.int32),
        jnp.cumsum((block_e[1:] != block_e[:-1]).astype(jnp.int32)),
    ])
    run_ext = jnp.concatenate([run_idx, run_idx[-1:]])

    xs, dest = _make_dispatch(s, cap)(flat, e_arr, hist)

    ys = _grouped_ffn(xs, Wg, Wu, Wd, be_ext, run_ext, nreal, nb)

    out = _make_combine(s, cap)(ys, dest, w_arr)
    return out.reshape(bsz, seq, d)


# speedup vs baseline: 1.0420x; 1.0420x over previous
import functools

import jax
import jax.numpy as jnp
from jax import lax
from jax.experimental import pallas as pl
from jax.experimental.pallas import tpu as pltpu
from jax.experimental.pallas import tpu_sc as plsc

D_MODEL = 1024
HIDDEN = 2816
NE = 8
NW = 32
ROW_BLOCK = 256
RB_SHIFT = ROW_BLOCK.bit_length() - 1


def _wid_base(t_per_w):
    cid = lax.axis_index("c")
    sid = lax.axis_index("s")
    wid = sid * 2 + cid
    return wid, wid * t_per_w


def _make_route(s_tokens):
    T = s_tokens // NW
    mesh = plsc.VectorSubcoreMesh(core_axis_name="c", subcore_axis_name="s")

    @functools.partial(
        pl.kernel, mesh=mesh,
        compiler_params=pltpu.CompilerParams(needs_layout_passes=False),
        out_type=[
            jax.ShapeDtypeStruct((2, s_tokens), jnp.int32),
            jax.ShapeDtypeStruct((2, s_tokens), jnp.float32),
            jax.ShapeDtypeStruct((NW, 16), jnp.int32),
        ],
        scratch_types=[
            pltpu.VMEM((NE, T), jnp.float32),
            pltpu.VMEM((2, T), jnp.int32),
            pltpu.VMEM((2, T), jnp.float32),
            pltpu.VMEM((16,), jnp.int32),
        ],
    )
    def route(lt_hbm, e_hbm, w_hbm, hist_hbm, lt_v, e_v, w_v, hist_v):
        wid, base = _wid_base(T)
        for e in range(NE):
            pltpu.sync_copy(lt_hbm.at[e, pl.ds(base, T)], lt_v.at[e])
        lanes = lax.iota(jnp.int32, 16)
        hist = jnp.zeros((16,), jnp.int32)
        for ch in range(T // 16):
            sl = pl.ds(ch * 16, 16)
            ls = [lt_v[e, sl] for e in range(NE)]
            m1 = ls[0]
            i1 = jnp.zeros((16,), jnp.int32)
            neg = jnp.full((16,), -1e30, jnp.float32)
            for e in range(1, NE):
                gt = ls[e] > m1
                m1 = jnp.where(gt, ls[e], m1)
                i1 = jnp.where(gt, jnp.full((16,), e, jnp.int32), i1)
            m2 = neg
            i2 = jnp.zeros((16,), jnp.int32)
            for e in range(NE):
                cand = jnp.where(i1 == e, neg, ls[e])
                gt = cand > m2
                m2 = jnp.where(gt, cand, m2)
                i2 = jnp.where(gt, jnp.full((16,), e, jnp.int32), i2)
            w1 = 1.0 / (1.0 + jnp.exp(m2 - m1))
            e_v[0, sl] = i1
            e_v[1, sl] = i2
            w_v[0, sl] = w1
            w_v[1, sl] = 1.0 - w1
            zero16 = jnp.zeros((16,), jnp.int32)
            for e in range(NE):
                c1 = plsc.all_reduce_population_count(i1 == e)
                c2 = plsc.all_reduce_population_count(i2 == e)
                hist = hist + jnp.where(lanes == e, c1 + c2, zero16)
        hist_v[...] = hist
        for s_ in range(2):
            pltpu.sync_copy(e_v.at[s_], e_hbm.at[s_, pl.ds(base, T)])
            pltpu.sync_copy(w_v.at[s_], w_hbm.at[s_, pl.ds(base, T)])
        pltpu.sync_copy(hist_v, hist_hbm.at[wid])

    return route


def _make_dispatch(s_tokens, cap):
    T = s_tokens // NW
    mesh = plsc.VectorSubcoreMesh(core_axis_name="c", subcore_axis_name="s")

    @functools.partial(
        pl.kernel, mesh=mesh,
        compiler_params=pltpu.CompilerParams(needs_layout_passes=False),
        out_type=[
            jax.ShapeDtypeStruct((cap, D_MODEL), jnp.float32),
            jax.ShapeDtypeStruct((2, s_tokens), jnp.int32),
        ],
        scratch_types=[
            pltpu.VMEM((NW, 16), jnp.int32),
            pltpu.VMEM((2, T), jnp.int32),
            pltpu.VMEM((2, T), jnp.int32),
            pltpu.VMEM((2, 16, D_MODEL), jnp.float32),
            pltpu.SemaphoreType.DMA((2,)),
            pltpu.SemaphoreType.DMA((2,)),
            pltpu.SemaphoreType.DMA((2,)),
        ],
    )
    def dispatch(x_hbm, e_hbm, hist_hbm, xs_hbm, dest_hbm,
                 hist_v, e_v, d_v, xbuf, sem1, sem2, in_sem):
        wid, base = _wid_base(T)
        pltpu.sync_copy(hist_hbm, hist_v)
        for s_ in range(2):
            pltpu.sync_copy(e_hbm.at[s_, pl.ds(base, T)], e_v.at[s_])
        lanes = lax.iota(jnp.int32, 16)
        zero16 = jnp.zeros((16,), jnp.int32)
        wid_v = jnp.full((16,), wid, jnp.int32)
        counts = zero16
        pre = zero16
        for wp in range(NW):
            row = hist_v[wp]
            counts = counts + row
            before = jnp.full((16,), wp, jnp.int32) < wid_v
            pre = pre + jnp.where(before, row, zero16)
        pc = ((counts + (ROW_BLOCK - 1)) >> RB_SHIFT) << RB_SHIFT
        incl = plsc.cumsum(pc)
        base_vec = (incl - pc) + pre
        b = [jnp.full((16,), jnp.max(jnp.where(lanes == e, base_vec, zero16)),
                      jnp.int32)
             for e in range(NE)]
        r = [zero16 for _ in range(NE)]
        for s_ in range(2):
            for ch in range(T // 16):
                sl = pl.ds(ch * 16, 16)
                v = e_v[s_, sl]
                dch = zero16
                for e in range(NE):
                    m = v == e
                    csum = plsc.cumsum(m.astype(jnp.int32))
                    dch = jnp.where(m, b[e] + r[e] + csum - 1, dch)
                    r[e] = r[e] + plsc.all_reduce_population_count(m)
                d_v[s_, sl] = dch
        for s_ in range(2):
            pltpu.sync_copy(d_v.at[s_], dest_hbm.at[s_, pl.ds(base, T)])
        nch = T // 16
        in_cp = {}
        sc_cp = {}
        in_cp[0] = pltpu.async_copy(
            x_hbm.at[pl.ds(base, 16)], xbuf.at[0], in_sem.at[0])
        for ch in range(nch):
            bslot = ch % 2
            nbslot = 1 - bslot
            in_cp[ch].wait()
            sl = pl.ds(ch * 16, 16)
            sc_cp[ch] = (
                pltpu.async_copy(xbuf.at[bslot], xs_hbm.at[d_v[0, sl]],
                                 sem1.at[bslot]),
                pltpu.async_copy(xbuf.at[bslot], xs_hbm.at[d_v[1, sl]],
                                 sem2.at[bslot]),
            )
            if ch + 1 < nch:
                if ch >= 1:
                    for c in sc_cp[ch - 1]:
                        c.wait()
                in_cp[ch + 1] = pltpu.async_copy(
                    x_hbm.at[pl.ds(base + (ch + 1) * 16, 16)],
                    xbuf.at[nbslot], in_sem.at[nbslot])
        for c in sc_cp[nch - 2] + sc_cp[nch - 1]:
            c.wait()

    return dispatch


def _make_combine(s_tokens, cap):
    T = s_tokens // NW
    mesh = plsc.VectorSubcoreMesh(core_axis_name="c", subcore_axis_name="s")

    @functools.partial(
        pl.kernel, mesh=mesh,
        compiler_params=pltpu.CompilerParams(needs_layout_passes=False),
        out_type=jax.ShapeDtypeStruct((s_tokens, D_MODEL), jnp.float32),
        scratch_types=[
            pltpu.VMEM((2, T), jnp.int32),
            pltpu.VMEM((2, T), jnp.float32),
            pltpu.VMEM((2, 16, D_MODEL), jnp.float32),
            pltpu.VMEM((2, 16, D_MODEL), jnp.float32),
            pltpu.SemaphoreType.DMA((2,)),
            pltpu.SemaphoreType.DMA((2,)),
            pltpu.SemaphoreType.DMA((2,)),
        ],
    )
    def combine(ys_hbm, dest_hbm, w_hbm, out_hbm,
                d_v, w_v, y1, y2, sem1, sem2, out_sem):
        wid, base = _wid_base(T)
        for s_ in range(2):
            pltpu.sync_copy(dest_hbm.at[s_, pl.ds(base, T)], d_v.at[s_])
            pltpu.sync_copy(w_hbm.at[s_, pl.ds(base, T)], w_v.at[s_])
        nch = T // 16
        g_cp = {}
        o_cp = {}
        g_cp[0] = (
            pltpu.async_copy(ys_hbm.at[d_v[0, pl.ds(0, 16)]], y1.at[0],
                             sem1.at[0]),
            pltpu.async_copy(ys_hbm.at[d_v[1, pl.ds(0, 16)]], y2.at[0],
                             sem2.at[0]),
        )
        for ch in range(nch):
            bslot = ch % 2
            nbslot = 1 - bslot
            for c in g_cp[ch]:
                c.wait()
            if ch + 1 < nch:
                if ch >= 1:
                    o_cp[ch - 1].wait()
                sln = pl.ds((ch + 1) * 16, 16)
                g_cp[ch + 1] = (
                    pltpu.async_copy(ys_hbm.at[d_v[0, sln]], y1.at[nbslot],
                                     sem1.at[nbslot]),
                    pltpu.async_copy(ys_hbm.at[d_v[1, sln]], y2.at[nbslot],
                                     sem2.at[nbslot]),
                )
            sl = pl.ds(ch * 16, 16)
            wch1 = w_v[0, sl]
            wch2 = w_v[1, sl]
            w1s = [jnp.full((16,), wch1[t], jnp.float32) for t in range(16)]
            w2s = [jnp.full((16,), wch2[t], jnp.float32) for t in range(16)]

            def blk_fn(blk, _, bslot=bslot, w1s=w1s, w2s=w2s):
                bsl = pl.ds(blk * 16, 16)
                for t in range(16):
                    y1[bslot, t, bsl] = (w1s[t] * y1[bslot, t, bsl]
                                         + w2s[t] * y2[bslot, t, bsl])
                return 0

            lax.fori_loop(0, D_MODEL // 16, blk_fn, 0)
            o_cp[ch] = pltpu.async_copy(
                y1.at[bslot], out_hbm.at[pl.ds(base + ch * 16, 16)],
                out_sem.at[bslot])
        o_cp[nch - 2].wait()
        o_cp[nch - 1].wait()

    return combine


def _ffn_body(be_ref, run_ref, nreal_ref,
              xs_ref, wg_hbm, wu_hbm, wd_hbm, out_ref,
              wgb, wub, wdb, g_sem, u_sem, d_sem):
    i = pl.program_id(0)
    slot = 0

    def copies(e, s):
        return (
            pltpu.make_async_copy(wg_hbm.at[e], wgb.at[s], g_sem.at[s]),
            pltpu.make_async_copy(wu_hbm.at[e], wub.at[s], u_sem.at[s]),
            pltpu.make_async_copy(wd_hbm.at[e], wdb.at[s], d_sem.at[s]),
        )

    @pl.when((i == 0) | (run_ref[i] != run_ref[jnp.maximum(i - 1, 0)]))
    def _():
        for c in copies(be_ref[i], 0):
            c.start()
        for c in copies(be_ref[i], 0):
            c.wait()

    @pl.when(i < nreal_ref[0])
    def _():
        x = xs_ref[...]
        g = jax.lax.dot_general(x, wgb[slot], (((1,), (1,)), ((), ())),
                                preferred_element_type=jnp.float32)
        u = jax.lax.dot_general(x, wub[slot], (((1,), (1,)), ((), ())),
                                preferred_element_type=jnp.float32)
        h = g * jax.nn.sigmoid(g) * u
        out_ref[...] = jax.lax.dot_general(h, wdb[slot], (((1,), (1,)), ((), ())),
                                           preferred_element_type=jnp.float32)


def _grouped_ffn(xs, wg, wu, wd, block_e, run_idx, nreal, nb):
    grid_spec = pltpu.PrefetchScalarGridSpec(
        num_scalar_prefetch=3,
        grid=(nb,),
        in_specs=[
            pl.BlockSpec((ROW_BLOCK, D_MODEL), lambda i, be, run, nr: (i, 0)),
            pl.BlockSpec(memory_space=pltpu.HBM),
            pl.BlockSpec(memory_space=pltpu.HBM),
            pl.BlockSpec(memory_space=pltpu.HBM),
        ],
        out_specs=pl.BlockSpec((ROW_BLOCK, D_MODEL), lambda i, be, run, nr: (i, 0)),
        scratch_shapes=[
            pltpu.VMEM((1, HIDDEN, D_MODEL), jnp.float32),
            pltpu.VMEM((1, HIDDEN, D_MODEL), jnp.float32),
            pltpu.VMEM((1, D_MODEL, HIDDEN), jnp.float32),
            pltpu.SemaphoreType.DMA((2,)),
            pltpu.SemaphoreType.DMA((2,)),
            pltpu.SemaphoreType.DMA((2,)),
        ],
    )
    return pl.pallas_call(
        _ffn_body,
        grid_spec=grid_spec,
        out_shape=jax.ShapeDtypeStruct((xs.shape[0], D_MODEL), jnp.float32),
        compiler_params=pltpu.CompilerParams(
            dimension_semantics=("arbitrary",),
        ),
    )(block_e, run_idx, nreal, xs, wg, wu, wd)


def kernel(x, W_router, Wg, Wu, Wd):
    bsz, seq, d = x.shape
    flat = x.reshape(-1, d)
    s = flat.shape[0]
    cap = s * 2 + NE * ROW_BLOCK
    nb = cap // ROW_BLOCK

    logits = flat @ W_router.T
    lt = logits.T

    e_arr, w_arr, hist = _make_route(s)(lt)

    counts = jnp.sum(hist, axis=0)[:NE]
    nblocks_e = (counts + ROW_BLOCK - 1) // ROW_BLOCK
    block_e = jnp.sum(
        (jnp.arange(nb)[:, None] >= jnp.cumsum(nblocks_e)[None, :]), axis=1
    ).astype(jnp.int32)
    block_e = jnp.minimum(block_e, NE - 1)
    nreal = jnp.sum(nblocks_e).astype(jnp.int32).reshape((1,))
    be_ext = jnp.concatenate([block_e, block_e[-1:]])
    run_idx = jnp.concatenate([
        jnp.zeros((1,), jnp.int32),
        jnp.cumsum((block_e[1:] != block_e[:-1]).astype(jnp.int32)),
    ])
    run_ext = jnp.concatenate([run_idx, run_idx[-1:]])

    xs, dest = _make_dispatch(s, cap)(flat, e_arr, hist)

    ys = _grouped_ffn(xs, Wg, Wu, Wd, be_ext, run_ext, nreal, nb)

    out = _make_combine(s, cap)(ys, dest, w_arr)
    return out.reshape(bsz, seq, d)


# speedup vs baseline: 1.0524x; 1.0100x over previous
import functools

import jax
import jax.numpy as jnp
from jax import lax
from jax.experimental import pallas as pl
from jax.experimental.pallas import tpu as pltpu
from jax.experimental.pallas import tpu_sc as plsc

D_MODEL = 1024
HIDDEN = 2816
NE = 8
NW = 32
ROW_BLOCK = 256
RB_SHIFT = ROW_BLOCK.bit_length() - 1


def _wid_base(t_per_w):
    cid = lax.axis_index("c")
    sid = lax.axis_index("s")
    wid = sid * 2 + cid
    return wid, wid * t_per_w


def _make_route(s_tokens):
    T = s_tokens // NW
    mesh = plsc.VectorSubcoreMesh(core_axis_name="c", subcore_axis_name="s")

    @functools.partial(
        pl.kernel, mesh=mesh,
        compiler_params=pltpu.CompilerParams(needs_layout_passes=False),
        out_type=[
            jax.ShapeDtypeStruct((2, s_tokens), jnp.int32),
            jax.ShapeDtypeStruct((2, s_tokens), jnp.float32),
            jax.ShapeDtypeStruct((NW, 16), jnp.int32),
        ],
        scratch_types=[
            pltpu.VMEM((NE, T), jnp.float32),
            pltpu.VMEM((2, T), jnp.int32),
            pltpu.VMEM((2, T), jnp.float32),
            pltpu.VMEM((16,), jnp.int32),
        ],
    )
    def route(lt_hbm, e_hbm, w_hbm, hist_hbm, lt_v, e_v, w_v, hist_v):
        wid, base = _wid_base(T)
        for e in range(NE):
            pltpu.sync_copy(lt_hbm.at[e, pl.ds(base, T)], lt_v.at[e])
        lanes = lax.iota(jnp.int32, 16)
        hist = jnp.zeros((16,), jnp.int32)
        for ch in range(T // 16):
            sl = pl.ds(ch * 16, 16)
            ls = [lt_v[e, sl] for e in range(NE)]
            m1 = ls[0]
            i1 = jnp.zeros((16,), jnp.int32)
            neg = jnp.full((16,), -1e30, jnp.float32)
            for e in range(1, NE):
                gt = ls[e] > m1
                m1 = jnp.where(gt, ls[e], m1)
                i1 = jnp.where(gt, jnp.full((16,), e, jnp.int32), i1)
            m2 = neg
            i2 = jnp.zeros((16,), jnp.int32)
            for e in range(NE):
                cand = jnp.where(i1 == e, neg, ls[e])
                gt = cand > m2
                m2 = jnp.where(gt, cand, m2)
                i2 = jnp.where(gt, jnp.full((16,), e, jnp.int32), i2)
            w1 = 1.0 / (1.0 + jnp.exp(m2 - m1))
            e_v[0, sl] = i1
            e_v[1, sl] = i2
            w_v[0, sl] = w1
            w_v[1, sl] = 1.0 - w1
            zero16 = jnp.zeros((16,), jnp.int32)
            for e in range(NE):
                c1 = plsc.all_reduce_population_count(i1 == e)
                c2 = plsc.all_reduce_population_count(i2 == e)
                hist = hist + jnp.where(lanes == e, c1 + c2, zero16)
        hist_v[...] = hist
        for s_ in range(2):
            pltpu.sync_copy(e_v.at[s_], e_hbm.at[s_, pl.ds(base, T)])
            pltpu.sync_copy(w_v.at[s_], w_hbm.at[s_, pl.ds(base, T)])
        pltpu.sync_copy(hist_v, hist_hbm.at[wid])

    return route


def _make_dispatch(s_tokens, cap):
    T = s_tokens // NW
    mesh = plsc.VectorSubcoreMesh(core_axis_name="c", subcore_axis_name="s")

    @functools.partial(
        pl.kernel, mesh=mesh,
        compiler_params=pltpu.CompilerParams(needs_layout_passes=False),
        out_type=[
            jax.ShapeDtypeStruct((cap, D_MODEL), jnp.float32),
            jax.ShapeDtypeStruct((2, s_tokens), jnp.int32),
        ],
        scratch_types=[
            pltpu.VMEM((NW, 16), jnp.int32),
            pltpu.VMEM((2, T), jnp.int32),
            pltpu.VMEM((2, T), jnp.int32),
            pltpu.VMEM((2, 16, D_MODEL), jnp.float32),
            pltpu.SemaphoreType.DMA((2,)),
            pltpu.SemaphoreType.DMA((2,)),
            pltpu.SemaphoreType.DMA((2,)),
        ],
    )
    def dispatch(x_hbm, e_hbm, hist_hbm, xs_hbm, dest_hbm,
                 hist_v, e_v, d_v, xbuf, sem1, sem2, in_sem):
        wid, base = _wid_base(T)
        pltpu.sync_copy(hist_hbm, hist_v)
        for s_ in range(2):
            pltpu.sync_copy(e_hbm.at[s_, pl.ds(base, T)], e_v.at[s_])
        lanes = lax.iota(jnp.int32, 16)
        zero16 = jnp.zeros((16,), jnp.int32)
        wid_v = jnp.full((16,), wid, jnp.int32)
        counts = zero16
        pre = zero16
        for wp in range(NW):
            row = hist_v[wp]
            counts = counts + row
            before = jnp.full((16,), wp, jnp.int32) < wid_v
            pre = pre + jnp.where(before, row, zero16)
        pc = ((counts + (ROW_BLOCK - 1)) >> RB_SHIFT) << RB_SHIFT
        incl = plsc.cumsum(pc)
        base_vec = (incl - pc) + pre
        b = [jnp.full((16,), jnp.max(jnp.where(lanes == e, base_vec, zero16)),
                      jnp.int32)
             for e in range(NE)]
        r = [zero16 for _ in range(NE)]
        for s_ in range(2):
            for ch in range(T // 16):
                sl = pl.ds(ch * 16, 16)
                v = e_v[s_, sl]
                dch = zero16
                for e in range(NE):
                    m = v == e
                    csum = plsc.cumsum(m.astype(jnp.int32))
                    dch = jnp.where(m, b[e] + r[e] + csum - 1, dch)
                    r[e] = r[e] + plsc.all_reduce_population_count(m)
                d_v[s_, sl] = dch
        for s_ in range(2):
            pltpu.sync_copy(d_v.at[s_], dest_hbm.at[s_, pl.ds(base, T)])
        nch = T // 16
        in_cp = {}
        sc_cp = {}
        in_cp[0] = pltpu.async_copy(
            x_hbm.at[pl.ds(base, 16)], xbuf.at[0], in_sem.at[0])
        for ch in range(nch):
            bslot = ch % 2
            nbslot = 1 - bslot
            in_cp[ch].wait()
            sl = pl.ds(ch * 16, 16)
            sc_cp[ch] = (
                pltpu.async_copy(xbuf.at[bslot], xs_hbm.at[d_v[0, sl]],
                                 sem1.at[bslot]),
                pltpu.async_copy(xbuf.at[bslot], xs_hbm.at[d_v[1, sl]],
                                 sem2.at[bslot]),
            )
            if ch + 1 < nch:
                if ch >= 1:
                    for c in sc_cp[ch - 1]:
                        c.wait()
                in_cp[ch + 1] = pltpu.async_copy(
                    x_hbm.at[pl.ds(base + (ch + 1) * 16, 16)],
                    xbuf.at[nbslot], in_sem.at[nbslot])
        for c in sc_cp[nch - 2] + sc_cp[nch - 1]:
            c.wait()

    return dispatch


def _make_combine(s_tokens, cap):
    T = s_tokens // NW
    mesh = plsc.VectorSubcoreMesh(core_axis_name="c", subcore_axis_name="s")

    @functools.partial(
        pl.kernel, mesh=mesh,
        compiler_params=pltpu.CompilerParams(needs_layout_passes=False),
        out_type=jax.ShapeDtypeStruct((s_tokens, D_MODEL), jnp.float32),
        scratch_types=[
            pltpu.VMEM((2, T), jnp.int32),
            pltpu.VMEM((2, T), jnp.float32),
            pltpu.VMEM((2, 16, D_MODEL), jnp.float32),
            pltpu.VMEM((2, 16, D_MODEL), jnp.float32),
            pltpu.SemaphoreType.DMA((2,)),
            pltpu.SemaphoreType.DMA((2,)),
            pltpu.SemaphoreType.DMA((2,)),
        ],
    )
    def combine(ys_hbm, dest_hbm, w_hbm, out_hbm,
                d_v, w_v, y1, y2, sem1, sem2, out_sem):
        wid, base = _wid_base(T)
        for s_ in range(2):
            pltpu.sync_copy(dest_hbm.at[s_, pl.ds(base, T)], d_v.at[s_])
            pltpu.sync_copy(w_hbm.at[s_, pl.ds(base, T)], w_v.at[s_])
        nch = T // 16
        g_cp = {}
        o_cp = {}
        g_cp[0] = (
            pltpu.async_copy(ys_hbm.at[d_v[0, pl.ds(0, 16)]], y1.at[0],
                             sem1.at[0]),
            pltpu.async_copy(ys_hbm.at[d_v[1, pl.ds(0, 16)]], y2.at[0],
                             sem2.at[0]),
        )
        for ch in range(nch):
            bslot = ch % 2
            nbslot = 1 - bslot
            for c in g_cp[ch]:
                c.wait()
            if ch + 1 < nch:
                if ch >= 1:
                    o_cp[ch - 1].wait()
                sln = pl.ds((ch + 1) * 16, 16)
                g_cp[ch + 1] = (
                    pltpu.async_copy(ys_hbm.at[d_v[0, sln]], y1.at[nbslot],
                                     sem1.at[nbslot]),
                    pltpu.async_copy(ys_hbm.at[d_v[1, sln]], y2.at[nbslot],
                                     sem2.at[nbslot]),
                )
            sl = pl.ds(ch * 16, 16)
            wch1 = w_v[0, sl]
            wch2 = w_v[1, sl]
            w1s = [jnp.full((16,), wch1[t], jnp.float32) for t in range(16)]
            w2s = [jnp.full((16,), wch2[t], jnp.float32) for t in range(16)]

            def blk_fn(blk, _, bslot=bslot, w1s=w1s, w2s=w2s):
                bsl = pl.ds(blk * 16, 16)
                for t in range(16):
                    y1[bslot, t, bsl] = (w1s[t] * y1[bslot, t, bsl]
                                         + w2s[t] * y2[bslot, t, bsl])
                return 0

            lax.fori_loop(0, D_MODEL // 16, blk_fn, 0)
            o_cp[ch] = pltpu.async_copy(
                y1.at[bslot], out_hbm.at[pl.ds(base + ch * 16, 16)],
                out_sem.at[bslot])
        o_cp[nch - 2].wait()
        o_cp[nch - 1].wait()

    return combine


def _ffn_body(be_ref, run_ref, nreal_ref,
              xs_ref, wg_hbm, wu_hbm, wd_hbm, out_ref,
              wgb, wub, wdb, g_sem, u_sem, d_sem):
    i = pl.program_id(0)
    slot = 0

    def copies(e, s):
        return (
            pltpu.make_async_copy(wg_hbm.at[e], wgb.at[s], g_sem.at[s]),
            pltpu.make_async_copy(wu_hbm.at[e], wub.at[s], u_sem.at[s]),
            pltpu.make_async_copy(wd_hbm.at[e], wdb.at[s], d_sem.at[s]),
        )

    first = (i == 0) | (run_ref[i] != run_ref[jnp.maximum(i - 1, 0)])
    live = i < nreal_ref[0]

    @pl.when(first & live)
    def _():
        for c in copies(be_ref[i], 0):
            c.start()

    @pl.when(live)
    def _():
        cg, cu, cd = copies(be_ref[i], 0)

        @pl.when(first)
        def _():
            cg.wait()

        x = xs_ref[...]
        g = jax.lax.dot_general(x, wgb[slot], (((1,), (1,)), ((), ())),
                                preferred_element_type=jnp.float32)

        @pl.when(first)
        def _():
            cu.wait()

        u = jax.lax.dot_general(x, wub[slot], (((1,), (1,)), ((), ())),
                                preferred_element_type=jnp.float32)
        h = g * jax.nn.sigmoid(g) * u

        @pl.when(first)
        def _():
            cd.wait()

        out_ref[...] = jax.lax.dot_general(h, wdb[slot], (((1,), (1,)), ((), ())),
                                           preferred_element_type=jnp.float32)


def _grouped_ffn(xs, wg, wu, wd, block_e, run_idx, nreal, nb):
    grid_spec = pltpu.PrefetchScalarGridSpec(
        num_scalar_prefetch=3,
        grid=(nb,),
        in_specs=[
            pl.BlockSpec((ROW_BLOCK, D_MODEL), lambda i, be, run, nr: (i, 0)),
            pl.BlockSpec(memory_space=pltpu.HBM),
            pl.BlockSpec(memory_space=pltpu.HBM),
            pl.BlockSpec(memory_space=pltpu.HBM),
        ],
        out_specs=pl.BlockSpec((ROW_BLOCK, D_MODEL), lambda i, be, run, nr: (i, 0)),
        scratch_shapes=[
            pltpu.VMEM((1, HIDDEN, D_MODEL), jnp.float32),
            pltpu.VMEM((1, HIDDEN, D_MODEL), jnp.float32),
            pltpu.VMEM((1, D_MODEL, HIDDEN), jnp.float32),
            pltpu.SemaphoreType.DMA((2,)),
            pltpu.SemaphoreType.DMA((2,)),
            pltpu.SemaphoreType.DMA((2,)),
        ],
    )
    return pl.pallas_call(
        _ffn_body,
        grid_spec=grid_spec,
        out_shape=jax.ShapeDtypeStruct((xs.shape[0], D_MODEL), jnp.float32),
        compiler_params=pltpu.CompilerParams(
            dimension_semantics=("arbitrary",),
        ),
    )(block_e, run_idx, nreal, xs, wg, wu, wd)


def kernel(x, W_router, Wg, Wu, Wd):
    bsz, seq, d = x.shape
    flat = x.reshape(-1, d)
    s = flat.shape[0]
    cap = s * 2 + NE * ROW_BLOCK
    nb = cap // ROW_BLOCK

    logits = flat @ W_router.T
    lt = logits.T

    e_arr, w_arr, hist = _make_route(s)(lt)

    counts = jnp.sum(hist, axis=0)[:NE]
    nblocks_e = (counts + ROW_BLOCK - 1) // ROW_BLOCK
    block_e = jnp.sum(
        (jnp.arange(nb)[:, None] >= jnp.cumsum(nblocks_e)[None, :]), axis=1
    ).astype(jnp.int32)
    block_e = jnp.minimum(block_e, NE - 1)
    nreal = jnp.sum(nblocks_e).astype(jnp.int32).reshape((1,))
    be_ext = jnp.concatenate([block_e, block_e[-1:]])
    run_idx = jnp.concatenate([
        jnp.zeros((1,), jnp.int32),
        jnp.cumsum((block_e[1:] != block_e[:-1]).astype(jnp.int32)),
    ])
    run_ext = jnp.concatenate([run_idx, run_idx[-1:]])

    xs, dest = _make_dispatch(s, cap)(flat, e_arr, hist)

    ys = _grouped_ffn(xs, Wg, Wu, Wd, be_ext, run_ext, nreal, nb)

    out = _make_combine(s, cap)(ys, dest, w_arr)
    return out.reshape(bsz, seq, d)
